# Initial kernel scaffold; baseline (speedup 1.0000x reference)
import jax
import jax.numpy as jnp
from jax.experimental import pallas as pl


def kernel(feat, view, W1, b1, W2, b2):
    raise NotImplementedError("write your pallas kernel here")


# speedup vs baseline: 12.3843x; 12.3843x over previous
import functools

import jax
import jax.numpy as jnp
from jax import lax
from jax.experimental import pallas as pl
from jax.experimental.pallas import tpu as pltpu
from jax.experimental.pallas import tpu_sc as plsc

N_NODES = 10000
E_EDGES = 320000
F_IN = 128
HID = 256
C_CLS = 40

NP = 10240
RPT = NP // 16
G = 128
NG = E_EDGES // G
BN = 256

_MESH = plsc.VectorSubcoreMesh(core_axis_name="c", subcore_axis_name="s")


def _deg_body(view3, deg_out, dstv, onesv, buf, deg_sp):
    c = lax.axis_index("c")
    s = lax.axis_index("s")
    for i in range(G // 16):
        onesv[pl.ds(i * 16, 16)] = jnp.ones((16,), jnp.float32)
    for i in range(RPT // 16):
        buf[pl.ds(i * 16, 16)] = jnp.zeros((16,), jnp.float32)
    pltpu.sync_copy(buf, deg_sp.at[pl.ds(s * RPT, RPT)])
    plsc.subcore_barrier()

    half = NG // 2
    base = c * half
    g0 = base + s * 78 + jnp.minimum(s, 2)
    g1 = base + (s + 1) * 78 + jnp.minimum(s + 1, 2)

    def body(g, carry):
        pltpu.sync_copy(view3.at[1, g], dstv)
        pltpu.sync_copy(onesv, deg_sp.at[dstv], add=True)
        return carry

    lax.fori_loop(g0, g1, body, 0)
    plsc.subcore_barrier()
    pltpu.sync_copy(deg_sp.at[pl.ds(s * RPT, RPT)], buf)
    pltpu.sync_copy(buf, deg_out.at[c, pl.ds(s * RPT, RPT)])


_deg_call = pl.kernel(
    _deg_body,
    out_type=jax.ShapeDtypeStruct((2, NP), jnp.float32),
    mesh=_MESH,
    scratch_types=[
        pltpu.VMEM((G,), jnp.int32),
        pltpu.VMEM((G,), jnp.float32),
        pltpu.VMEM((RPT,), jnp.float32),
        pltpu.VMEM_SHARED((NP,), jnp.float32),
    ],
)


def _agg_body(W, hs2d, view3, out, srcv, dstv, gidxv, buf, agg_sp):
    c = lax.axis_index("c")
    s = lax.axis_index("s")
    off = c * NP
    for k in range(RPT // G):
        r = s * RPT + k * G
        pltpu.sync_copy(hs2d.at[pl.ds(off + r, G)], buf)
        pltpu.sync_copy(buf, agg_sp.at[pl.ds(r, G)])
    plsc.subcore_barrier()

    g0 = s * 156 + jnp.minimum(s, 4)
    g1 = (s + 1) * 156 + jnp.minimum(s + 1, 4)

    def body(g, carry):
        pltpu.sync_copy(view3.at[0, g], srcv)
        pltpu.sync_copy(view3.at[1, g], dstv)
        for i in range(G // 16):
            gidxv[pl.ds(i * 16, 16)] = srcv[pl.ds(i * 16, 16)] + off
        pltpu.sync_copy(hs2d.at[gidxv], buf)
        pltpu.sync_copy(buf, agg_sp.at[dstv], add=True)
        return carry

    lax.fori_loop(g0, g1, body, 0)
    plsc.subcore_barrier()
    for k in range(RPT // G):
        r = s * RPT + k * G
        pltpu.sync_copy(agg_sp.at[pl.ds(r, G)], buf)
        pltpu.sync_copy(buf, out.at[c, pl.ds(r, G)])


_agg1_call = pl.kernel(
    functools.partial(_agg_body, HID // 2),
    out_type=jax.ShapeDtypeStruct((2, NP, HID // 2), jnp.float32),
    mesh=_MESH,
    scratch_types=[
        pltpu.VMEM((G,), jnp.int32),
        pltpu.VMEM((G,), jnp.int32),
        pltpu.VMEM((G,), jnp.int32),
        pltpu.VMEM((G, HID // 2), jnp.float32),
        pltpu.VMEM_SHARED((NP, HID // 2), jnp.float32),
    ],
)


def _agg2_body(hs2p, view3, out, srcv, dstv, buf, acc_sp):
    c = lax.axis_index("c")
    s = lax.axis_index("s")

    for k in range(RPT // G):
        r = s * RPT + k * G
        pltpu.sync_copy(hs2p.at[pl.ds(r, G)], buf)
        pltpu.sync_copy(buf, acc_sp.at[pl.ds(r, G)])

    plsc.subcore_barrier()

    half = NG // 2
    g0 = c * half + s * 78 + jnp.minimum(s, 2)
    g1 = c * half + (s + 1) * 78 + jnp.minimum(s + 1, 2)

    def body(g, carry):
        pltpu.sync_copy(view3.at[0, g], srcv)
        pltpu.sync_copy(view3.at[1, g], dstv)
        pltpu.sync_copy(hs2p.at[srcv], buf)
        pltpu.sync_copy(buf, acc_sp.at[dstv], add=True)
        return carry

    lax.fori_loop(g0, g1, body, 0)
    plsc.subcore_barrier()
    for k in range(RPT // G):
        r = s * RPT + k * G
        pltpu.sync_copy(acc_sp.at[pl.ds(r, G)], buf)
        pltpu.sync_copy(buf, out.at[c, pl.ds(r, G)])


_agg2_call = pl.kernel(
    _agg2_body,
    out_type=jax.ShapeDtypeStruct((2, NP, 128), jnp.float32),
    mesh=_MESH,
    scratch_types=[
        pltpu.VMEM((G,), jnp.int32),
        pltpu.VMEM((G,), jnp.int32),
        pltpu.VMEM((G, 128), jnp.float32),
        pltpu.VMEM_SHARED((NP, 128), jnp.float32),
    ],
)


def _mm1_body(feat_ref, w1_ref, deg_ref, hs_ref, dinv_ref):
    d = deg_ref[0, :] + deg_ref[1, :] + 1.0
    dinv = lax.rsqrt(d)[:, None]
    h = jnp.dot(feat_ref[...], w1_ref[...],
                preferred_element_type=jnp.float32,
                precision=lax.Precision.HIGHEST)
    hs = h * dinv
    hs_ref[0] = hs[:, : HID // 2]
    hs_ref[1] = hs[:, HID // 2:]
    dinv_ref[...] = dinv


_mm1_call = pl.pallas_call(
    _mm1_body,
    grid=(NP // BN,),
    in_specs=[
        pl.BlockSpec((BN, F_IN), lambda i: (i, 0)),
        pl.BlockSpec((F_IN, HID), lambda i: (0, 0)),
        pl.BlockSpec((2, BN), lambda i: (0, i)),
    ],
    out_specs=[
        pl.BlockSpec((2, BN, HID // 2), lambda i: (0, i, 0)),
        pl.BlockSpec((BN, 1), lambda i: (i, 0)),
    ],
    out_shape=[
        jax.ShapeDtypeStruct((2, NP, HID // 2), jnp.float32),
        jax.ShapeDtypeStruct((NP, 1), jnp.float32),
    ],
)


def _mm2_body(agg_ref, dinv_ref, b1_ref, w2_ref, out_ref):
    x = jnp.concatenate([agg_ref[0], agg_ref[1]], axis=1)
    dinv = dinv_ref[...]
    h = jnp.maximum(x * dinv + b1_ref[...], 0.0)
    h2 = jnp.dot(h, w2_ref[...],
                 preferred_element_type=jnp.float32,
                 precision=lax.Precision.HIGHEST)
    out_ref[...] = h2 * dinv


_mm2_call = pl.pallas_call(
    _mm2_body,
    grid=(NP // BN,),
    in_specs=[
        pl.BlockSpec((2, BN, HID // 2), lambda i: (0, i, 0)),
        pl.BlockSpec((BN, 1), lambda i: (i, 0)),
        pl.BlockSpec((1, HID), lambda i: (0, 0)),
        pl.BlockSpec((HID, 128), lambda i: (0, 0)),
    ],
    out_specs=pl.BlockSpec((BN, 128), lambda i: (i, 0)),
    out_shape=jax.ShapeDtypeStruct((NP, 128), jnp.float32),
)


def _head_body(agg_ref, hs2_ref, dinv_ref, b2_ref, out_ref):
    x = agg_ref[0, :, :64] + agg_ref[1, :, :64] - hs2_ref[:, :64]
    logits = x * dinv_ref[...] + b2_ref[...]
    m = jnp.max(logits, axis=1, keepdims=True)
    e = jnp.exp(logits - m)
    out_ref[...] = e / jnp.sum(e, axis=1, keepdims=True)


_head_call = pl.pallas_call(
    _head_body,
    grid=(NP // BN,),
    in_specs=[
        pl.BlockSpec((2, BN, 128), lambda i: (0, i, 0)),
        pl.BlockSpec((BN, 128), lambda i: (i, 0)),
        pl.BlockSpec((BN, 1), lambda i: (i, 0)),
        pl.BlockSpec((1, 64), lambda i: (0, 0)),
    ],
    out_specs=pl.BlockSpec((BN, 64), lambda i: (i, 0)),
    out_shape=jax.ShapeDtypeStruct((NP, 64), jnp.float32),
)


def kernel(feat, view, W1, b1, W2, b2):
    featp = jnp.zeros((NP, F_IN), jnp.float32).at[:N_NODES].set(feat)
    view3 = view.reshape(2, NG, G)

    deg2 = _deg_call(view3)
    hs1, dinv = _mm1_call(featp, W1, deg2)
    agg1 = _agg1_call(hs1.reshape(2 * NP, HID // 2), view3)

    w2p = jnp.zeros((HID, 128), jnp.float32).at[:, :C_CLS].set(W2)
    hs2 = _mm2_call(agg1, dinv, b1.reshape(1, HID), w2p)
    agg2 = _agg2_call(hs2, view3)

    b2p = jnp.full((1, 64), -1e30, jnp.float32).at[0, :C_CLS].set(b2)
    prob = _head_call(agg2, hs2, dinv, b2p)
    return prob[:N_NODES, :C_CLS]


# speedup vs baseline: 19.3301x; 1.5609x over previous
import functools

import jax
import jax.numpy as jnp
from jax import lax
from jax.experimental import pallas as pl
from jax.experimental.pallas import tpu as pltpu
from jax.experimental.pallas import tpu_sc as plsc

N_NODES = 10000
E_EDGES = 320000
F_IN = 128
HID = 256
C_CLS = 40

NP = 10240
RPT = NP // 16
G = 128
NGP = 2560
EP = NGP * G
BN = 256

_MESH = plsc.VectorSubcoreMesh(core_axis_name="c", subcore_axis_name="s")


def _deg_body(didx2, deg_out, dstv, onesv, buf, deg_sp):
    c = lax.axis_index("c")
    s = lax.axis_index("s")
    for i in range(G // 16):
        onesv[pl.ds(i * 16, 16)] = jnp.ones((16,), jnp.float32)
    for i in range(RPT // 16):
        buf[pl.ds(i * 16, 16)] = jnp.zeros((16,), jnp.float32)
    pltpu.sync_copy(buf, deg_sp.at[pl.ds(s * RPT, RPT)])
    plsc.subcore_barrier()

    gpt = NGP // 32
    g0 = c * (NGP // 2) + s * gpt

    def body(g, carry):
        pltpu.sync_copy(didx2.at[g0 + g], dstv)
        pltpu.sync_copy(onesv, deg_sp.at[dstv], add=True)
        return carry

    lax.fori_loop(0, gpt, body, 0)
    plsc.subcore_barrier()
    pltpu.sync_copy(deg_sp.at[pl.ds(s * RPT, RPT)], buf)
    pltpu.sync_copy(buf, deg_out.at[c, pl.ds(s * RPT, RPT)])


_deg_call = pl.kernel(
    _deg_body,
    out_type=jax.ShapeDtypeStruct((2, NP), jnp.float32),
    mesh=_MESH,
    scratch_types=[
        pltpu.VMEM((G,), jnp.int32),
        pltpu.VMEM((G,), jnp.float32),
        pltpu.VMEM((RPT,), jnp.float32),
        pltpu.VMEM_SHARED((NP,), jnp.float32),
    ],
)


CH = 16


def _agg_body(gpt, edge_split, table, sidx3, didx2, out,
              si0, si1, di0, di1, buf0, buf1,
              semi0, semi1, semg0, semg1, sems0, sems1, acc_sp):
    nch = gpt // CH
    c = lax.axis_index("c")
    s = lax.axis_index("s")
    if edge_split:
        gr0 = c * (NGP // 2) + s * gpt
        seed_off = 0
        plane = 0
    else:
        gr0 = s * gpt
        seed_off = c * NP
        plane = c

    sib = (si0, si1)
    dib = (di0, di1)
    bufs = (buf0, buf1)
    semis = (semi0, semi1)
    semgs = (semg0, semg1)
    semss = (sems0, sems1)

    def i_start(ch, p):
        pltpu.async_copy(
            sidx3.at[plane, pl.ds(gr0 + ch * CH, CH)], sib[p], semis[p])
        pltpu.async_copy(
            didx2.at[pl.ds(gr0 + ch * CH, CH)], dib[p], semis[p])

    def i_wait(p):
        pltpu.make_async_copy(
            sidx3.at[0, pl.ds(0, CH)], sib[p], semis[p]).wait()
        pltpu.make_async_copy(
            didx2.at[pl.ds(0, CH)], dib[p], semis[p]).wait()

    def g_start(j, b, p):
        pltpu.async_copy(table.at[sib[p].at[j]], bufs[b], semgs[b])

    def g_wait(b):
        pltpu.make_async_copy(table.at[pl.ds(0, G)], bufs[b], semgs[b]).wait()

    def s_start(j, b, p):
        pltpu.async_copy(bufs[b], acc_sp.at[dib[p].at[j]], semss[b], add=True)

    def s_wait(j, b, p):
        pltpu.make_async_copy(
            bufs[b], acc_sp.at[dib[p].at[j]], semss[b]).wait()

    i_start(0, 0)
    for k in range(RPT // G):
        r = s * RPT + k * G
        pltpu.sync_copy(table.at[pl.ds(seed_off + r, G)], buf0)
        pltpu.sync_copy(buf0, acc_sp.at[pl.ds(r, G)])
    plsc.subcore_barrier()

    for ch in range(nch):
        p = ch % 2
        i_wait(p)
        if ch > 0:
            s_wait(CH - 1, 1, 1 - p)
        if ch + 1 < nch:
            i_start(ch + 1, 1 - p)
        g_start(0, 0, p)
        g_wait(0)
        g_start(1, 1, p)
        s_start(0, 0, p)

        def inner(j2, carry, p=p):
            jj = 2 * j2 + 1
            g_wait(1)
            s_wait(jj - 1, 0, p)
            g_start(jj + 1, 0, p)
            s_start(jj, 1, p)
            g_wait(0)
            s_wait(jj, 1, p)
            g_start(jj + 2, 1, p)
            s_start(jj + 1, 0, p)
            return carry

        lax.fori_loop(0, (CH - 2) // 2, inner, 0)
        g_wait(1)
        s_wait(CH - 2, 0, p)
        s_start(CH - 1, 1, p)

    s_wait(CH - 1, 1, (nch - 1) % 2)
    plsc.subcore_barrier()
    for k in range(RPT // G):
        r = s * RPT + k * G
        pltpu.sync_copy(acc_sp.at[pl.ds(r, G)], buf0)
        pltpu.sync_copy(buf0, out.at[c, pl.ds(r, G)])


def _make_agg_call(gpt, edge_split):
    return pl.kernel(
        functools.partial(_agg_body, gpt, edge_split),
        out_type=jax.ShapeDtypeStruct((2, NP, 128), jnp.float32),
        mesh=_MESH,
        scratch_types=[
            pltpu.VMEM((CH, G), jnp.int32),
            pltpu.VMEM((CH, G), jnp.int32),
            pltpu.VMEM((CH, G), jnp.int32),
            pltpu.VMEM((CH, G), jnp.int32),
            pltpu.VMEM((G, 128), jnp.float32),
            pltpu.VMEM((G, 128), jnp.float32),
            pltpu.SemaphoreType.DMA,
            pltpu.SemaphoreType.DMA,
            pltpu.SemaphoreType.DMA,
            pltpu.SemaphoreType.DMA,
            pltpu.SemaphoreType.DMA,
            pltpu.SemaphoreType.DMA,
            pltpu.VMEM_SHARED((NP, 128), jnp.float32),
        ],
    )


_agg1_call = _make_agg_call(NGP // 16, False)
_agg2_call = _make_agg_call(NGP // 32, True)


def _mm1_body(feat_ref, w1_ref, deg_ref, hs_ref, dinv_ref):
    d = deg_ref[0, :] + deg_ref[1, :] + 1.0
    dinv = lax.rsqrt(d)[:, None]
    h = jnp.dot(feat_ref[...], w1_ref[...],
                preferred_element_type=jnp.float32,
                precision=lax.Precision.HIGHEST)
    hs = h * dinv
    hs_ref[0] = hs[:, : HID // 2]
    hs_ref[1] = hs[:, HID // 2:]
    dinv_ref[...] = dinv


_mm1_call = pl.pallas_call(
    _mm1_body,
    grid=(NP // BN,),
    in_specs=[
        pl.BlockSpec((BN, F_IN), lambda i: (i, 0)),
        pl.BlockSpec((F_IN, HID), lambda i: (0, 0)),
        pl.BlockSpec((2, BN), lambda i: (0, i)),
    ],
    out_specs=[
        pl.BlockSpec((2, BN, HID // 2), lambda i: (0, i, 0)),
        pl.BlockSpec((BN, 1), lambda i: (i, 0)),
    ],
    out_shape=[
        jax.ShapeDtypeStruct((2, NP, HID // 2), jnp.float32),
        jax.ShapeDtypeStruct((NP, 1), jnp.float32),
    ],
)


def _mm2_body(agg_ref, dinv_ref, b1_ref, w2_ref, out_ref):
    x = jnp.concatenate([agg_ref[0], agg_ref[1]], axis=1)
    dinv = dinv_ref[...]
    h = jnp.maximum(x * dinv + b1_ref[...], 0.0)
    h2 = jnp.dot(h, w2_ref[...],
                 preferred_element_type=jnp.float32,
                 precision=lax.Precision.HIGHEST)
    out_ref[...] = h2 * dinv


_mm2_call = pl.pallas_call(
    _mm2_body,
    grid=(NP // BN,),
    in_specs=[
        pl.BlockSpec((2, BN, HID // 2), lambda i: (0, i, 0)),
        pl.BlockSpec((BN, 1), lambda i: (i, 0)),
        pl.BlockSpec((1, HID), lambda i: (0, 0)),
        pl.BlockSpec((HID, 128), lambda i: (0, 0)),
    ],
    out_specs=pl.BlockSpec((BN, 128), lambda i: (i, 0)),
    out_shape=jax.ShapeDtypeStruct((NP, 128), jnp.float32),
)


def _head_body(agg_ref, hs2_ref, dinv_ref, b2_ref, out_ref):
    x = agg_ref[0, :, :64] + agg_ref[1, :, :64] - hs2_ref[:, :64]
    logits = x * dinv_ref[...] + b2_ref[...]
    m = jnp.max(logits, axis=1, keepdims=True)
    e = jnp.exp(logits - m)
    out_ref[...] = e / jnp.sum(e, axis=1, keepdims=True)


_head_call = pl.pallas_call(
    _head_body,
    grid=(NP // BN,),
    in_specs=[
        pl.BlockSpec((2, BN, 128), lambda i: (0, i, 0)),
        pl.BlockSpec((BN, 128), lambda i: (i, 0)),
        pl.BlockSpec((BN, 1), lambda i: (i, 0)),
        pl.BlockSpec((1, 64), lambda i: (0, 0)),
    ],
    out_specs=pl.BlockSpec((BN, 64), lambda i: (i, 0)),
    out_shape=jax.ShapeDtypeStruct((NP, 64), jnp.float32),
)


def kernel(feat, view, W1, b1, W2, b2):
    featp = jnp.zeros((NP, F_IN), jnp.float32).at[:N_NODES].set(feat)

    npad = EP - E_EDGES
    pad_i = jnp.arange(npad, dtype=jnp.int32)
    src = jnp.concatenate([view[0], pad_i % N_NODES])
    dst = jnp.concatenate([view[1], N_NODES + pad_i % (NP - N_NODES)])
    sidx3 = jnp.stack([src, src + NP]).reshape(2, NGP, G)
    didx2 = dst.reshape(NGP, G)

    deg2 = _deg_call(didx2)
    hs1, dinv = _mm1_call(featp, W1, deg2)
    agg1 = _agg1_call(hs1.reshape(2 * NP, HID // 2), sidx3, didx2)

    w2p = jnp.zeros((HID, 128), jnp.float32).at[:, :C_CLS].set(W2)
    hs2 = _mm2_call(agg1, dinv, b1.reshape(1, HID), w2p)
    agg2 = _agg2_call(hs2, sidx3, didx2)

    b2p = jnp.full((1, 64), -1e30, jnp.float32).at[0, :C_CLS].set(b2)
    prob = _head_call(agg2, hs2, dinv, b2p)
    return prob[:N_NODES, :C_CLS]


# speedup vs baseline: 20.7040x; 1.0711x over previous
import functools

import jax
import jax.numpy as jnp
from jax import lax
from jax.experimental import pallas as pl
from jax.experimental.pallas import tpu as pltpu
from jax.experimental.pallas import tpu_sc as plsc

N_NODES = 10000
E_EDGES = 320000
F_IN = 128
HID = 256
C_CLS = 40

NP = 10240
RPT = NP // 16
G = 128
NGP = 2560
CH = 16
EP = NGP * G
BN = 256

_MESH = plsc.VectorSubcoreMesh(core_axis_name="c", subcore_axis_name="s")


def _deg_body(didx2, deg_out, di0, di1, onesv, buf,
              semi0, semi1, sems0, sems1, deg_sp):
    c = lax.axis_index("c")
    s = lax.axis_index("s")
    gpt = NGP // 32
    nch = gpt // CH
    g0 = c * (NGP // 2) + s * gpt
    dib = (di0, di1)
    semis = (semi0, semi1)
    semss = (sems0, sems1)

    def i_start(ch, p):
        pltpu.async_copy(didx2.at[pl.ds(g0 + ch * CH, CH)], dib[p], semis[p])

    for i in range(G // 16):
        onesv[pl.ds(i * 16, 16)] = jnp.ones((16,), jnp.float32)
    i_start(0, 0)
    for i in range(RPT // 16):
        buf[pl.ds(i * 16, 16)] = jnp.zeros((16,), jnp.float32)
    pltpu.sync_copy(buf, deg_sp.at[pl.ds(s * RPT, RPT)])
    plsc.subcore_barrier()

    for ch in range(nch):
        p = ch % 2
        pltpu.make_async_copy(didx2.at[pl.ds(0, CH)], dib[p], semis[p]).wait()
        if ch > 0:
            for j in range(CH):
                pltpu.make_async_copy(
                    onesv, deg_sp.at[dib[1 - p].at[j]], semss[1 - p]).wait()
        if ch + 1 < nch:
            i_start(ch + 1, 1 - p)
        for j in range(CH):
            pltpu.async_copy(onesv, deg_sp.at[dib[p].at[j]], semss[p],
                             add=True)
    for j in range(CH):
        pltpu.make_async_copy(
            onesv, deg_sp.at[dib[(nch - 1) % 2].at[j]],
            semss[(nch - 1) % 2]).wait()
    plsc.subcore_barrier()
    pltpu.sync_copy(deg_sp.at[pl.ds(s * RPT, RPT)], buf)
    pltpu.sync_copy(buf, deg_out.at[c, pl.ds(s * RPT, RPT)])


_deg_call = pl.kernel(
    _deg_body,
    out_type=jax.ShapeDtypeStruct((2, NP), jnp.float32),
    mesh=_MESH,
    scratch_types=[
        pltpu.VMEM((CH, G), jnp.int32),
        pltpu.VMEM((CH, G), jnp.int32),
        pltpu.VMEM((G,), jnp.float32),
        pltpu.VMEM((RPT,), jnp.float32),
        pltpu.SemaphoreType.DMA,
        pltpu.SemaphoreType.DMA,
        pltpu.SemaphoreType.DMA,
        pltpu.SemaphoreType.DMA,
        pltpu.VMEM_SHARED((NP,), jnp.float32),
    ],
)


def _agg_body(gpt, edge_split, table, sidx3, didx2, out,
              si0, si1, di0, di1, buf0, buf1,
              semi0, semi1, semg0, semg1, sems0, sems1, acc_sp):
    nch = gpt // CH
    c = lax.axis_index("c")
    s = lax.axis_index("s")
    if edge_split:
        gr0 = c * (NGP // 2) + s * gpt
        seed_off = 0
        plane = 0
    else:
        gr0 = s * gpt
        seed_off = c * NP
        plane = c

    sib = (si0, si1)
    dib = (di0, di1)
    bufs = (buf0, buf1)
    semis = (semi0, semi1)
    semgs = (semg0, semg1)
    semss = (sems0, sems1)

    def i_start(ch, p):
        pltpu.async_copy(
            sidx3.at[plane, pl.ds(gr0 + ch * CH, CH)], sib[p], semis[p])
        pltpu.async_copy(
            didx2.at[pl.ds(gr0 + ch * CH, CH)], dib[p], semis[p])

    def i_wait(p):
        pltpu.make_async_copy(
            sidx3.at[0, pl.ds(0, CH)], sib[p], semis[p]).wait()
        pltpu.make_async_copy(
            didx2.at[pl.ds(0, CH)], dib[p], semis[p]).wait()

    def g_start(j, b, p):
        pltpu.async_copy(table.at[sib[p].at[j]], bufs[b], semgs[b])

    def g_wait(b):
        pltpu.make_async_copy(table.at[pl.ds(0, G)], bufs[b], semgs[b]).wait()

    def s_start(j, b, p):
        pltpu.async_copy(bufs[b], acc_sp.at[dib[p].at[j]], semss[b], add=True)

    def s_wait(j, b, p):
        pltpu.make_async_copy(
            bufs[b], acc_sp.at[dib[p].at[j]], semss[b]).wait()

    i_start(0, 0)
    for k in range(RPT // G):
        r = s * RPT + k * G
        pltpu.sync_copy(table.at[pl.ds(seed_off + r, G)], buf0)
        pltpu.sync_copy(buf0, acc_sp.at[pl.ds(r, G)])
    plsc.subcore_barrier()

    for ch in range(nch):
        p = ch % 2
        i_wait(p)
        if ch > 0:
            s_wait(CH - 1, 1, 1 - p)
        if ch + 1 < nch:
            i_start(ch + 1, 1 - p)
        g_start(0, 0, p)
        g_wait(0)
        g_start(1, 1, p)
        s_start(0, 0, p)

        def inner(j2, carry, p=p):
            jj = 2 * j2 + 1
            g_wait(1)
            s_wait(jj - 1, 0, p)
            g_start(jj + 1, 0, p)
            s_start(jj, 1, p)
            g_wait(0)
            s_wait(jj, 1, p)
            g_start(jj + 2, 1, p)
            s_start(jj + 1, 0, p)
            return carry

        lax.fori_loop(0, (CH - 2) // 2, inner, 0)
        g_wait(1)
        s_wait(CH - 2, 0, p)
        s_start(CH - 1, 1, p)

    s_wait(CH - 1, 1, (nch - 1) % 2)
    plsc.subcore_barrier()
    for k in range(RPT // G):
        r = s * RPT + k * G
        pltpu.sync_copy(acc_sp.at[pl.ds(r, G)], buf0)
        pltpu.sync_copy(buf0, out.at[c, pl.ds(r, G)])


def _make_agg_call(gpt, edge_split):
    return pl.kernel(
        functools.partial(_agg_body, gpt, edge_split),
        out_type=jax.ShapeDtypeStruct((2, NP, 128), jnp.float32),
        mesh=_MESH,
        scratch_types=[
            pltpu.VMEM((CH, G), jnp.int32),
            pltpu.VMEM((CH, G), jnp.int32),
            pltpu.VMEM((CH, G), jnp.int32),
            pltpu.VMEM((CH, G), jnp.int32),
            pltpu.VMEM((G, 128), jnp.float32),
            pltpu.VMEM((G, 128), jnp.float32),
            pltpu.SemaphoreType.DMA,
            pltpu.SemaphoreType.DMA,
            pltpu.SemaphoreType.DMA,
            pltpu.SemaphoreType.DMA,
            pltpu.SemaphoreType.DMA,
            pltpu.SemaphoreType.DMA,
            pltpu.VMEM_SHARED((NP, 128), jnp.float32),
        ],
    )


_agg1_call = _make_agg_call(NGP // 16, False)
_agg2_call = _make_agg_call(NGP // 32, True)


def _mm1_body(feat_ref, w1_ref, deg_ref, hs_ref, dinv_ref):
    d = deg_ref[0, :] + deg_ref[1, :] + 1.0
    dinv = lax.rsqrt(d)[:, None]
    h = jnp.dot(feat_ref[...], w1_ref[...],
                preferred_element_type=jnp.float32,
                precision=lax.Precision.HIGHEST)
    hs = h * dinv
    hs_ref[0] = hs[:, : HID // 2]
    hs_ref[1] = hs[:, HID // 2:]
    dinv_ref[...] = dinv


_mm1_call = pl.pallas_call(
    _mm1_body,
    grid=(NP // BN,),
    in_specs=[
        pl.BlockSpec((BN, F_IN), lambda i: (i, 0)),
        pl.BlockSpec((F_IN, HID), lambda i: (0, 0)),
        pl.BlockSpec((2, BN), lambda i: (0, i)),
    ],
    out_specs=[
        pl.BlockSpec((2, BN, HID // 2), lambda i: (0, i, 0)),
        pl.BlockSpec((BN, 1), lambda i: (i, 0)),
    ],
    out_shape=[
        jax.ShapeDtypeStruct((2, NP, HID // 2), jnp.float32),
        jax.ShapeDtypeStruct((NP, 1), jnp.float32),
    ],
)


def _mm2_body(agg_ref, dinv_ref, b1_ref, w2_ref, out_ref):
    x = jnp.concatenate([agg_ref[0], agg_ref[1]], axis=1)
    dinv = dinv_ref[...]
    h = jnp.maximum(x * dinv + b1_ref[...], 0.0)
    h2 = jnp.dot(h, w2_ref[...],
                 preferred_element_type=jnp.float32,
                 precision=lax.Precision.HIGHEST)
    out_ref[...] = h2 * dinv


_mm2_call = pl.pallas_call(
    _mm2_body,
    grid=(NP // BN,),
    in_specs=[
        pl.BlockSpec((2, BN, HID // 2), lambda i: (0, i, 0)),
        pl.BlockSpec((BN, 1), lambda i: (i, 0)),
        pl.BlockSpec((1, HID), lambda i: (0, 0)),
        pl.BlockSpec((HID, 128), lambda i: (0, 0)),
    ],
    out_specs=pl.BlockSpec((BN, 128), lambda i: (i, 0)),
    out_shape=jax.ShapeDtypeStruct((NP, 128), jnp.float32),
)


def _head_body(agg_ref, hs2_ref, dinv_ref, b2_ref, out_ref):
    x = agg_ref[0, :, :64] + agg_ref[1, :, :64] - hs2_ref[:, :64]
    logits = x * dinv_ref[...] + b2_ref[...]
    m = jnp.max(logits, axis=1, keepdims=True)
    e = jnp.exp(logits - m)
    out_ref[...] = e / jnp.sum(e, axis=1, keepdims=True)


_head_call = pl.pallas_call(
    _head_body,
    grid=(NP // BN,),
    in_specs=[
        pl.BlockSpec((2, BN, 128), lambda i: (0, i, 0)),
        pl.BlockSpec((BN, 128), lambda i: (i, 0)),
        pl.BlockSpec((BN, 1), lambda i: (i, 0)),
        pl.BlockSpec((1, 64), lambda i: (0, 0)),
    ],
    out_specs=pl.BlockSpec((BN, 64), lambda i: (i, 0)),
    out_shape=jax.ShapeDtypeStruct((NP, 64), jnp.float32),
)


def kernel(feat, view, W1, b1, W2, b2):
    featp = jnp.zeros((NP, F_IN), jnp.float32).at[:N_NODES].set(feat)

    npad = EP - E_EDGES
    pad_i = jnp.arange(npad, dtype=jnp.int32)
    src = jnp.concatenate([view[0], pad_i % N_NODES])
    dst = jnp.concatenate([view[1], N_NODES + pad_i % (NP - N_NODES)])
    sidx3 = jnp.stack([src, src + NP]).reshape(2, NGP, G)
    didx2 = dst.reshape(NGP, G)

    deg2 = _deg_call(didx2)
    hs1, dinv = _mm1_call(featp, W1, deg2)
    agg1 = _agg1_call(hs1.reshape(2 * NP, HID // 2), sidx3, didx2)

    w2p = jnp.zeros((HID, 128), jnp.float32).at[:, :C_CLS].set(W2)
    hs2 = _mm2_call(agg1, dinv, b1.reshape(1, HID), w2p)
    agg2 = _agg2_call(hs2, sidx3, didx2)

    b2p = jnp.full((1, 64), -1e30, jnp.float32).at[0, :C_CLS].set(b2)
    prob = _head_call(agg2, hs2, dinv, b2p)
    return prob[:N_NODES, :C_CLS]


# speedup vs baseline: 24.0391x; 1.1611x over previous
import functools

import jax
import jax.numpy as jnp
from jax import lax
from jax.experimental import pallas as pl
from jax.experimental.pallas import tpu as pltpu
from jax.experimental.pallas import tpu_sc as plsc

N_NODES = 10000
E_EDGES = 320000
F_IN = 128
HID = 256
C_CLS = 40

NP = 10240
RPT = NP // 16
G = 128
NGP = 2560
CH = 16
EP = NGP * G
BN = 256

_MESH = plsc.VectorSubcoreMesh(core_axis_name="c", subcore_axis_name="s")


def _deg_body(didx2, deg_out, di0, di1, onesv, buf,
              semi0, semi1, sems0, sems1, deg_sp):
    c = lax.axis_index("c")
    s = lax.axis_index("s")
    gpt = NGP // 32
    nch = gpt // CH
    g0 = c * (NGP // 2) + s * gpt
    dib = (di0, di1)
    semis = (semi0, semi1)
    semss = (sems0, sems1)

    def i_start(ch, p):
        pltpu.async_copy(didx2.at[pl.ds(g0 + ch * CH, CH)], dib[p], semis[p])

    for i in range(G // 16):
        onesv[pl.ds(i * 16, 16)] = jnp.ones((16,), jnp.float32)
    i_start(0, 0)
    for i in range(RPT // 16):
        buf[pl.ds(i * 16, 16)] = jnp.zeros((16,), jnp.float32)
    pltpu.sync_copy(buf, deg_sp.at[pl.ds(s * RPT, RPT)])
    plsc.subcore_barrier()

    for ch in range(nch):
        p = ch % 2
        pltpu.make_async_copy(didx2.at[pl.ds(0, CH)], dib[p], semis[p]).wait()
        if ch > 0:
            for j in range(CH):
                pltpu.make_async_copy(
                    onesv, deg_sp.at[dib[1 - p].at[j]], semss[1 - p]).wait()
        if ch + 1 < nch:
            i_start(ch + 1, 1 - p)
        for j in range(CH):
            pltpu.async_copy(onesv, deg_sp.at[dib[p].at[j]], semss[p],
                             add=True)
    for j in range(CH):
        pltpu.make_async_copy(
            onesv, deg_sp.at[dib[(nch - 1) % 2].at[j]],
            semss[(nch - 1) % 2]).wait()
    plsc.subcore_barrier()
    pltpu.sync_copy(deg_sp.at[pl.ds(s * RPT, RPT)], buf)
    pltpu.sync_copy(buf, deg_out.at[c, pl.ds(s * RPT, RPT)])


_deg_call = pl.kernel(
    _deg_body,
    out_type=jax.ShapeDtypeStruct((2, NP), jnp.float32),
    mesh=_MESH,
    scratch_types=[
        pltpu.VMEM((CH, G), jnp.int32),
        pltpu.VMEM((CH, G), jnp.int32),
        pltpu.VMEM((G,), jnp.float32),
        pltpu.VMEM((RPT,), jnp.float32),
        pltpu.SemaphoreType.DMA,
        pltpu.SemaphoreType.DMA,
        pltpu.SemaphoreType.DMA,
        pltpu.SemaphoreType.DMA,
        pltpu.VMEM_SHARED((NP,), jnp.float32),
    ],
)


def _agg_body(gpt, edge_split, table, sidx3, didx2, out,
              si0, si1, di0, di1, buf0, buf1,
              semi0, semi1, semg0, semg1, sems0, sems1, acc_sp):
    nch = gpt // CH
    c = lax.axis_index("c")
    s = lax.axis_index("s")
    if edge_split:
        gr0 = c * (NGP // 2) + s * gpt
        seed_off = 0
        plane = 0
    else:
        gr0 = s * gpt
        seed_off = c * NP
        plane = c

    sib = (si0, si1)
    dib = (di0, di1)
    bufs = (buf0, buf1)
    semis = (semi0, semi1)
    semgs = (semg0, semg1)
    semss = (sems0, sems1)

    def i_start(ch, p):
        pltpu.async_copy(
            sidx3.at[plane, pl.ds(gr0 + ch * CH, CH)], sib[p], semis[p])
        pltpu.async_copy(
            didx2.at[pl.ds(gr0 + ch * CH, CH)], dib[p], semis[p])

    def i_wait(p):
        pltpu.make_async_copy(
            sidx3.at[0, pl.ds(0, CH)], sib[p], semis[p]).wait()
        pltpu.make_async_copy(
            didx2.at[pl.ds(0, CH)], dib[p], semis[p]).wait()

    def g_start(j, b, p):
        pltpu.async_copy(table.at[sib[p].at[j]], bufs[b], semgs[b])

    def g_wait(b):
        pltpu.make_async_copy(table.at[pl.ds(0, G)], bufs[b], semgs[b]).wait()

    def s_start(j, b, p):
        pltpu.async_copy(bufs[b], acc_sp.at[dib[p].at[j]], semss[b], add=True)

    def s_wait(j, b, p):
        pltpu.make_async_copy(
            bufs[b], acc_sp.at[dib[p].at[j]], semss[b]).wait()

    i_start(0, 0)
    for k in range(RPT // G):
        r = s * RPT + k * G
        pltpu.sync_copy(table.at[pl.ds(seed_off + r, G)], buf0)
        pltpu.sync_copy(buf0, acc_sp.at[pl.ds(r, G)])
    plsc.subcore_barrier()

    for ch in range(nch):
        p = ch % 2
        if ch == 0:
            i_wait(0)
            if nch > 1:
                i_start(1, 1)
            g_start(0, 0, 0)
            g_start(1, 1, 0)
            g_wait(0)
            s_start(0, 0, 0)
        else:
            s_wait(CH - 1, 1, 1 - p)
            i_start(ch + 1, 1 - p) if ch + 1 < nch else None
            g_start(1, 1, p)
            g_wait(0)
            s_start(0, 0, p)

        def inner(j2, carry, p=p):
            jj = 2 * j2 + 1
            s_wait(jj - 1, 0, p)
            g_start(jj + 1, 0, p)
            g_wait(1)
            s_start(jj, 1, p)
            s_wait(jj, 1, p)
            g_start(jj + 2, 1, p)
            g_wait(0)
            s_start(jj + 1, 0, p)
            return carry

        lax.fori_loop(0, (CH - 2) // 2, inner, 0)
        s_wait(CH - 2, 0, p)
        if ch + 1 < nch:
            i_wait(1 - p)
            g_start(0, 0, 1 - p)
        g_wait(1)
        s_start(CH - 1, 1, p)

    s_wait(CH - 1, 1, (nch - 1) % 2)
    plsc.subcore_barrier()
    for k in range(RPT // G):
        r = s * RPT + k * G
        pltpu.sync_copy(acc_sp.at[pl.ds(r, G)], buf0)
        pltpu.sync_copy(buf0, out.at[c, pl.ds(r, G)])


def _make_agg_call(gpt, edge_split):
    return pl.kernel(
        functools.partial(_agg_body, gpt, edge_split),
        out_type=jax.ShapeDtypeStruct((2, NP, 128), jnp.float32),
        mesh=_MESH,
        scratch_types=[
            pltpu.VMEM((CH, G), jnp.int32),
            pltpu.VMEM((CH, G), jnp.int32),
            pltpu.VMEM((CH, G), jnp.int32),
            pltpu.VMEM((CH, G), jnp.int32),
            pltpu.VMEM((G, 128), jnp.float32),
            pltpu.VMEM((G, 128), jnp.float32),
            pltpu.SemaphoreType.DMA,
            pltpu.SemaphoreType.DMA,
            pltpu.SemaphoreType.DMA,
            pltpu.SemaphoreType.DMA,
            pltpu.SemaphoreType.DMA,
            pltpu.SemaphoreType.DMA,
            pltpu.VMEM_SHARED((NP, 128), jnp.float32),
        ],
    )


_agg1_call = _make_agg_call(NGP // 16, False)
_agg2_call = _make_agg_call(NGP // 32, True)


def _mm1_body(feat_ref, w1_ref, deg_ref, hs_ref, dinv_ref):
    d = deg_ref[0, :] + deg_ref[1, :] + 1.0
    dinv = lax.rsqrt(d)[:, None]
    h = jnp.dot(feat_ref[...], w1_ref[...],
                preferred_element_type=jnp.float32,
                precision=lax.Precision.HIGHEST)
    hs = h * dinv
    hs_ref[0] = hs[:, : HID // 2]
    hs_ref[1] = hs[:, HID // 2:]
    dinv_ref[...] = dinv


_mm1_call = pl.pallas_call(
    _mm1_body,
    grid=(NP // BN,),
    in_specs=[
        pl.BlockSpec((BN, F_IN), lambda i: (i, 0)),
        pl.BlockSpec((F_IN, HID), lambda i: (0, 0)),
        pl.BlockSpec((2, BN), lambda i: (0, i)),
    ],
    out_specs=[
        pl.BlockSpec((2, BN, HID // 2), lambda i: (0, i, 0)),
        pl.BlockSpec((BN, 1), lambda i: (i, 0)),
    ],
    out_shape=[
        jax.ShapeDtypeStruct((2, NP, HID // 2), jnp.float32),
        jax.ShapeDtypeStruct((NP, 1), jnp.float32),
    ],
)


def _mm2_body(agg_ref, dinv_ref, b1_ref, w2_ref, out_ref):
    x = jnp.concatenate([agg_ref[0], agg_ref[1]], axis=1)
    dinv = dinv_ref[...]
    h = jnp.maximum(x * dinv + b1_ref[...], 0.0)
    h2 = jnp.dot(h, w2_ref[...],
                 preferred_element_type=jnp.float32,
                 precision=lax.Precision.HIGHEST)
    out_ref[...] = h2 * dinv


_mm2_call = pl.pallas_call(
    _mm2_body,
    grid=(NP // BN,),
    in_specs=[
        pl.BlockSpec((2, BN, HID // 2), lambda i: (0, i, 0)),
        pl.BlockSpec((BN, 1), lambda i: (i, 0)),
        pl.BlockSpec((1, HID), lambda i: (0, 0)),
        pl.BlockSpec((HID, 128), lambda i: (0, 0)),
    ],
    out_specs=pl.BlockSpec((BN, 128), lambda i: (i, 0)),
    out_shape=jax.ShapeDtypeStruct((NP, 128), jnp.float32),
)


def _head_body(agg_ref, hs2_ref, dinv_ref, b2_ref, out_ref):
    x = agg_ref[0, :, :64] + agg_ref[1, :, :64] - hs2_ref[:, :64]
    logits = x * dinv_ref[...] + b2_ref[...]
    m = jnp.max(logits, axis=1, keepdims=True)
    e = jnp.exp(logits - m)
    out_ref[...] = e / jnp.sum(e, axis=1, keepdims=True)


_head_call = pl.pallas_call(
    _head_body,
    grid=(NP // BN,),
    in_specs=[
        pl.BlockSpec((2, BN, 128), lambda i: (0, i, 0)),
        pl.BlockSpec((BN, 128), lambda i: (i, 0)),
        pl.BlockSpec((BN, 1), lambda i: (i, 0)),
        pl.BlockSpec((1, 64), lambda i: (0, 0)),
    ],
    out_specs=pl.BlockSpec((BN, 64), lambda i: (i, 0)),
    out_shape=jax.ShapeDtypeStruct((NP, 64), jnp.float32),
)


def kernel(feat, view, W1, b1, W2, b2):
    featp = jnp.zeros((NP, F_IN), jnp.float32).at[:N_NODES].set(feat)

    npad = EP - E_EDGES
    pad_i = jnp.arange(npad, dtype=jnp.int32)
    src = jnp.concatenate([view[0], pad_i % N_NODES])
    dst = jnp.concatenate([view[1], N_NODES + pad_i % (NP - N_NODES)])
    sidx3 = jnp.stack([src, src + NP]).reshape(2, NGP, G)
    didx2 = dst.reshape(NGP, G)

    deg2 = _deg_call(didx2)
    hs1, dinv = _mm1_call(featp, W1, deg2)
    agg1 = _agg1_call(hs1.reshape(2 * NP, HID // 2), sidx3, didx2)

    w2p = jnp.zeros((HID, 128), jnp.float32).at[:, :C_CLS].set(W2)
    hs2 = _mm2_call(agg1, dinv, b1.reshape(1, HID), w2p)
    agg2 = _agg2_call(hs2, sidx3, didx2)

    b2p = jnp.full((1, 64), -1e30, jnp.float32).at[0, :C_CLS].set(b2)
    prob = _head_call(agg2, hs2, dinv, b2p)
    return prob[:N_NODES, :C_CLS]


# speedup vs baseline: 24.9154x; 1.0365x over previous
import functools

import jax
import jax.numpy as jnp
from jax import lax
from jax.experimental import pallas as pl
from jax.experimental.pallas import tpu as pltpu
from jax.experimental.pallas import tpu_sc as plsc

N_NODES = 10000
E_EDGES = 320000
F_IN = 128
HID = 256
C_CLS = 40

NP = 10240
RPT = NP // 16
G = 128
NGP = 2560
CH = 16
EP = NGP * G
BN = 256

_MESH = plsc.VectorSubcoreMesh(core_axis_name="c", subcore_axis_name="s")


def _deg_body(didx2, deg_out, di0, di1, onesv, buf,
              semi0, semi1, sems0, sems1, deg_sp):
    c = lax.axis_index("c")
    s = lax.axis_index("s")
    gpt = NGP // 32
    nch = gpt // CH
    g0 = c * (NGP // 2) + s * gpt
    dib = (di0, di1)
    semis = (semi0, semi1)
    semss = (sems0, sems1)

    def i_start(ch, p):
        pltpu.async_copy(didx2.at[pl.ds(g0 + ch * CH, CH)], dib[p], semis[p])

    for i in range(G // 16):
        onesv[pl.ds(i * 16, 16)] = jnp.ones((16,), jnp.float32)
    i_start(0, 0)
    for i in range(RPT // 16):
        buf[pl.ds(i * 16, 16)] = jnp.zeros((16,), jnp.float32)
    pltpu.sync_copy(buf, deg_sp.at[pl.ds(s * RPT, RPT)])
    plsc.subcore_barrier()

    for ch in range(nch):
        p = ch % 2
        pltpu.make_async_copy(didx2.at[pl.ds(0, CH)], dib[p], semis[p]).wait()
        if ch > 0:
            for j in range(CH):
                pltpu.make_async_copy(
                    onesv, deg_sp.at[dib[1 - p].at[j]], semss[1 - p]).wait()
        if ch + 1 < nch:
            i_start(ch + 1, 1 - p)
        for j in range(CH):
            pltpu.async_copy(onesv, deg_sp.at[dib[p].at[j]], semss[p],
                             add=True)
    for j in range(CH):
        pltpu.make_async_copy(
            onesv, deg_sp.at[dib[(nch - 1) % 2].at[j]],
            semss[(nch - 1) % 2]).wait()
    plsc.subcore_barrier()
    pltpu.sync_copy(deg_sp.at[pl.ds(s * RPT, RPT)], buf)
    pltpu.sync_copy(buf, deg_out.at[c, pl.ds(s * RPT, RPT)])


_deg_call = pl.kernel(
    _deg_body,
    out_type=jax.ShapeDtypeStruct((2, NP), jnp.float32),
    mesh=_MESH,
    scratch_types=[
        pltpu.VMEM((CH, G), jnp.int32),
        pltpu.VMEM((CH, G), jnp.int32),
        pltpu.VMEM((G,), jnp.float32),
        pltpu.VMEM((RPT,), jnp.float32),
        pltpu.SemaphoreType.DMA,
        pltpu.SemaphoreType.DMA,
        pltpu.SemaphoreType.DMA,
        pltpu.SemaphoreType.DMA,
        pltpu.VMEM_SHARED((NP,), jnp.float32),
    ],
)


def _agg_body(gpt, edge_split, table, sidx3, didx2, out,
              si0, si1, di0, di1, buf0, buf1,
              semi0, semi1, semg0, semg1, sems0, sems1, acc_sp):
    nch = gpt // CH
    c = lax.axis_index("c")
    s = lax.axis_index("s")
    if edge_split:
        gr0 = c * (NGP // 2) + s * gpt
        seed_off = 0
        plane = 0
    else:
        gr0 = s * gpt
        seed_off = c * NP
        plane = c

    sib = (si0, si1)
    dib = (di0, di1)
    bufs = (buf0, buf1)
    semis = (semi0, semi1)
    semgs = (semg0, semg1)
    semss = (sems0, sems1)

    def i_start(ch, p):
        pltpu.async_copy(
            sidx3.at[plane, pl.ds(gr0 + ch * CH, CH)], sib[p], semis[p])
        pltpu.async_copy(
            didx2.at[pl.ds(gr0 + ch * CH, CH)], dib[p], semis[p])

    def i_wait(p):
        pltpu.make_async_copy(
            sidx3.at[0, pl.ds(0, CH)], sib[p], semis[p]).wait()
        pltpu.make_async_copy(
            didx2.at[pl.ds(0, CH)], dib[p], semis[p]).wait()

    def g_start(j, b, p):
        pltpu.async_copy(table.at[sib[p].at[j]], bufs[b], semgs[b])

    def g_wait(b):
        pltpu.make_async_copy(table.at[pl.ds(0, G)], bufs[b], semgs[b]).wait()

    def s_start(j, b, p):
        pltpu.async_copy(bufs[b], acc_sp.at[dib[p].at[j]], semss[b], add=True)

    def s_wait(j, b, p):
        pltpu.make_async_copy(
            bufs[b], acc_sp.at[dib[p].at[j]], semss[b]).wait()

    i_start(0, 0)
    for k in range(RPT // G):
        r = s * RPT + k * G
        pltpu.sync_copy(table.at[pl.ds(seed_off + r, G)], buf0)
        pltpu.sync_copy(buf0, acc_sp.at[pl.ds(r, G)])
    plsc.subcore_barrier()

    for ch in range(nch):
        p = ch % 2
        if ch == 0:
            i_wait(0)
            if nch > 1:
                i_start(1, 1)
            g_start(0, 0, 0)
            g_start(1, 1, 0)
            g_wait(0)
            s_start(0, 0, 0)
        else:
            s_wait(CH - 1, 1, 1 - p)
            i_start(ch + 1, 1 - p) if ch + 1 < nch else None
            g_start(1, 1, p)
            g_wait(0)
            s_start(0, 0, p)

        def inner(j2, carry, p=p):
            jj = 2 * j2 + 1
            s_wait(jj - 1, 0, p)
            g_start(jj + 1, 0, p)
            g_wait(1)
            s_start(jj, 1, p)
            s_wait(jj, 1, p)
            g_start(jj + 2, 1, p)
            g_wait(0)
            s_start(jj + 1, 0, p)
            return carry

        lax.fori_loop(0, (CH - 2) // 2, inner, 0)
        s_wait(CH - 2, 0, p)
        if ch + 1 < nch:
            i_wait(1 - p)
            g_start(0, 0, 1 - p)
        g_wait(1)
        s_start(CH - 1, 1, p)

    s_wait(CH - 1, 1, (nch - 1) % 2)
    plsc.subcore_barrier()
    for k in range(RPT // G):
        r = s * RPT + k * G
        pltpu.sync_copy(acc_sp.at[pl.ds(r, G)], buf0)
        pltpu.sync_copy(buf0, out.at[c, pl.ds(r, G)])


def _make_agg_call(gpt, edge_split):
    return pl.kernel(
        functools.partial(_agg_body, gpt, edge_split),
        out_type=jax.ShapeDtypeStruct((2, NP, 128), jnp.float32),
        mesh=_MESH,
        scratch_types=[
            pltpu.VMEM((CH, G), jnp.int32),
            pltpu.VMEM((CH, G), jnp.int32),
            pltpu.VMEM((CH, G), jnp.int32),
            pltpu.VMEM((CH, G), jnp.int32),
            pltpu.VMEM((G, 128), jnp.float32),
            pltpu.VMEM((G, 128), jnp.float32),
            pltpu.SemaphoreType.DMA,
            pltpu.SemaphoreType.DMA,
            pltpu.SemaphoreType.DMA,
            pltpu.SemaphoreType.DMA,
            pltpu.SemaphoreType.DMA,
            pltpu.SemaphoreType.DMA,
            pltpu.VMEM_SHARED((NP, 128), jnp.float32),
        ],
    )


_agg1_call = _make_agg_call(NGP // 16, False)
_agg2_call = _make_agg_call(NGP // 32, True)


def _mm1_body(feat_ref, w1_ref, deg_ref, hs_ref, dinv_ref):
    d = deg_ref[0, :] + deg_ref[1, :] + 1.0
    dinv = lax.rsqrt(d)[:, None]
    h = jnp.dot(feat_ref[...], w1_ref[...],
                preferred_element_type=jnp.float32)
    hs = h * dinv
    hs_ref[0] = hs[:, : HID // 2]
    hs_ref[1] = hs[:, HID // 2:]
    dinv_ref[...] = dinv


_mm1_call = pl.pallas_call(
    _mm1_body,
    grid=(NP // BN,),
    in_specs=[
        pl.BlockSpec((BN, F_IN), lambda i: (i, 0)),
        pl.BlockSpec((F_IN, HID), lambda i: (0, 0)),
        pl.BlockSpec((2, BN), lambda i: (0, i)),
    ],
    out_specs=[
        pl.BlockSpec((2, BN, HID // 2), lambda i: (0, i, 0)),
        pl.BlockSpec((BN, 1), lambda i: (i, 0)),
    ],
    out_shape=[
        jax.ShapeDtypeStruct((2, NP, HID // 2), jnp.float32),
        jax.ShapeDtypeStruct((NP, 1), jnp.float32),
    ],
)


def _mm2_body(agg_ref, dinv_ref, b1_ref, w2_ref, out_ref):
    x = jnp.concatenate([agg_ref[0], agg_ref[1]], axis=1)
    dinv = dinv_ref[...]
    h = jnp.maximum(x * dinv + b1_ref[...], 0.0)
    h2 = jnp.dot(h, w2_ref[...],
                 preferred_element_type=jnp.float32)
    out_ref[...] = h2 * dinv


_mm2_call = pl.pallas_call(
    _mm2_body,
    grid=(NP // BN,),
    in_specs=[
        pl.BlockSpec((2, BN, HID // 2), lambda i: (0, i, 0)),
        pl.BlockSpec((BN, 1), lambda i: (i, 0)),
        pl.BlockSpec((1, HID), lambda i: (0, 0)),
        pl.BlockSpec((HID, 128), lambda i: (0, 0)),
    ],
    out_specs=pl.BlockSpec((BN, 128), lambda i: (i, 0)),
    out_shape=jax.ShapeDtypeStruct((NP, 128), jnp.float32),
)


def _head_body(agg_ref, hs2_ref, dinv_ref, b2_ref, out_ref):
    x = agg_ref[0, :, :64] + agg_ref[1, :, :64] - hs2_ref[:, :64]
    logits = x * dinv_ref[...] + b2_ref[...]
    m = jnp.max(logits, axis=1, keepdims=True)
    e = jnp.exp(logits - m)
    out_ref[...] = e / jnp.sum(e, axis=1, keepdims=True)


_head_call = pl.pallas_call(
    _head_body,
    grid=(NP // BN,),
    in_specs=[
        pl.BlockSpec((2, BN, 128), lambda i: (0, i, 0)),
        pl.BlockSpec((BN, 128), lambda i: (i, 0)),
        pl.BlockSpec((BN, 1), lambda i: (i, 0)),
        pl.BlockSpec((1, 64), lambda i: (0, 0)),
    ],
    out_specs=pl.BlockSpec((BN, 64), lambda i: (i, 0)),
    out_shape=jax.ShapeDtypeStruct((NP, 64), jnp.float32),
)


def kernel(feat, view, W1, b1, W2, b2):
    featp = jnp.zeros((NP, F_IN), jnp.float32).at[:N_NODES].set(feat)

    npad = EP - E_EDGES
    pad_i = jnp.arange(npad, dtype=jnp.int32)
    src = jnp.concatenate([view[0], pad_i % N_NODES])
    dst = jnp.concatenate([view[1], N_NODES + pad_i % (NP - N_NODES)])
    sidx3 = jnp.stack([src, src + NP]).reshape(2, NGP, G)
    didx2 = dst.reshape(NGP, G)

    deg2 = _deg_call(didx2)
    hs1, dinv = _mm1_call(featp, W1, deg2)
    agg1 = _agg1_call(hs1.reshape(2 * NP, HID // 2), sidx3, didx2)

    w2p = jnp.zeros((HID, 128), jnp.float32).at[:, :C_CLS].set(W2)
    hs2 = _mm2_call(agg1, dinv, b1.reshape(1, HID), w2p)
    agg2 = _agg2_call(hs2, sidx3, didx2)

    b2p = jnp.full((1, 64), -1e30, jnp.float32).at[0, :C_CLS].set(b2)
    prob = _head_call(agg2, hs2, dinv, b2p)
    return prob[:N_NODES, :C_CLS]


# speedup vs baseline: 31.6034x; 1.2684x over previous
import jax
import jax.numpy as jnp
from jax import lax
from jax.experimental import pallas as pl
from jax.experimental.pallas import tpu as pltpu
from jax.experimental.pallas import tpu_sc as plsc

N_NODES = 10000
E_EDGES = 320000
F_IN = 128
HID = 256
C_CLS = 40

NP = 10240
RPT = NP // 16
G = 128
NGP = 2560
CH = 16
EP = NGP * G
BN = 256

_MESH = plsc.VectorSubcoreMesh(core_axis_name="c", subcore_axis_name="s")


def _deg_body(didx2, deg_out, di0, di1, onesv, buf,
              semi0, semi1, sems0, sems1, deg_sp):
    c = lax.axis_index("c")
    s = lax.axis_index("s")
    gpt = NGP // 32
    nch = gpt // CH
    g0 = c * (NGP // 2) + s * gpt
    dib = (di0, di1)
    semis = (semi0, semi1)
    semss = (sems0, sems1)

    def i_start(ch, p):
        pltpu.async_copy(didx2.at[pl.ds(g0 + ch * CH, CH)], dib[p], semis[p])

    for i in range(G // 16):
        onesv[pl.ds(i * 16, 16)] = jnp.ones((16,), jnp.float32)
    i_start(0, 0)
    for i in range(RPT // 16):
        buf[pl.ds(i * 16, 16)] = jnp.zeros((16,), jnp.float32)
    pltpu.sync_copy(buf, deg_sp.at[pl.ds(s * RPT, RPT)])
    plsc.subcore_barrier()

    for ch in range(nch):
        p = ch % 2
        pltpu.make_async_copy(didx2.at[pl.ds(0, CH)], dib[p], semis[p]).wait()
        if ch > 0:
            for j in range(CH):
                pltpu.make_async_copy(
                    onesv, deg_sp.at[dib[1 - p].at[j]], semss[1 - p]).wait()
        if ch + 1 < nch:
            i_start(ch + 1, 1 - p)
        for j in range(CH):
            pltpu.async_copy(onesv, deg_sp.at[dib[p].at[j]], semss[p],
                             add=True)
    for j in range(CH):
        pltpu.make_async_copy(
            onesv, deg_sp.at[dib[(nch - 1) % 2].at[j]],
            semss[(nch - 1) % 2]).wait()
    plsc.subcore_barrier()
    pltpu.sync_copy(deg_sp.at[pl.ds(s * RPT, RPT)], buf)
    pltpu.sync_copy(buf, deg_out.at[c, pl.ds(s * RPT, RPT)])


_deg_call = pl.kernel(
    _deg_body,
    out_type=jax.ShapeDtypeStruct((2, NP), jnp.float32),
    mesh=_MESH,
    scratch_types=[
        pltpu.VMEM((CH, G), jnp.int32),
        pltpu.VMEM((CH, G), jnp.int32),
        pltpu.VMEM((G,), jnp.float32),
        pltpu.VMEM((RPT,), jnp.float32),
        pltpu.SemaphoreType.DMA,
        pltpu.SemaphoreType.DMA,
        pltpu.SemaphoreType.DMA,
        pltpu.SemaphoreType.DMA,
        pltpu.VMEM_SHARED((NP,), jnp.float32),
    ],
)


def _agg_body(table, sidx2, didx2, out,
              si0, si1, di0, di1, buf0, buf1,
              semi0, semi1, semg0, semg1, sems0, sems1, acc_sp):
    gpt = NGP // 32
    nch = gpt // CH
    c = lax.axis_index("c")
    s = lax.axis_index("s")
    gr0 = c * (NGP // 2) + s * gpt

    sib = (si0, si1)
    dib = (di0, di1)
    bufs = (buf0, buf1)
    semis = (semi0, semi1)
    semgs = (semg0, semg1)
    semss = (sems0, sems1)

    def i_start(ch, p):
        pltpu.async_copy(
            sidx2.at[pl.ds(gr0 + ch * CH, CH)], sib[p], semis[p])
        pltpu.async_copy(
            didx2.at[pl.ds(gr0 + ch * CH, CH)], dib[p], semis[p])

    def i_wait(p):
        pltpu.make_async_copy(
            sidx2.at[pl.ds(0, CH)], sib[p], semis[p]).wait()
        pltpu.make_async_copy(
            didx2.at[pl.ds(0, CH)], dib[p], semis[p]).wait()

    def g_start(j, b, p):
        pltpu.async_copy(table.at[sib[p].at[j]], bufs[b], semgs[b])

    def g_wait(b):
        pltpu.make_async_copy(table.at[pl.ds(0, G)], bufs[b], semgs[b]).wait()

    def s_start(j, b, p):
        pltpu.async_copy(bufs[b], acc_sp.at[dib[p].at[j]], semss[b], add=True)

    def s_wait(j, b, p):
        pltpu.make_async_copy(
            bufs[b], acc_sp.at[dib[p].at[j]], semss[b]).wait()

    i_start(0, 0)
    nk = RPT // G

    def seed_in(k, b):
        pltpu.async_copy(
            table.at[pl.ds(s * RPT + k * G, G)], bufs[b], semgs[b])

    def seed_out(k, b):
        pltpu.async_copy(bufs[b], acc_sp.at[pl.ds(s * RPT + k * G, G)],
                         semss[b])

    def seed_out_wait(k, b):
        pltpu.make_async_copy(
            bufs[b], acc_sp.at[pl.ds(s * RPT + k * G, G)], semss[b]).wait()

    seed_in(0, 0)
    for k in range(nk):
        b = k % 2
        if k + 1 < nk:
            if k >= 1:
                seed_out_wait(k - 1, 1 - b)
            seed_in(k + 1, 1 - b)
        g_wait(b)
        seed_out(k, b)
    seed_out_wait(nk - 2, (nk - 2) % 2)
    seed_out_wait(nk - 1, (nk - 1) % 2)
    plsc.subcore_barrier()

    for ch in range(nch):
        p = ch % 2
        if ch == 0:
            i_wait(0)
            if nch > 1:
                i_start(1, 1)
            g_start(0, 0, 0)
            g_start(1, 1, 0)
            g_wait(0)
            s_start(0, 0, 0)
        else:
            s_wait(CH - 1, 1, 1 - p)
            i_start(ch + 1, 1 - p) if ch + 1 < nch else None
            g_start(1, 1, p)
            g_wait(0)
            s_start(0, 0, p)

        def inner(j2, carry, p=p):
            jj = 2 * j2 + 1
            s_wait(jj - 1, 0, p)
            g_start(jj + 1, 0, p)
            g_wait(1)
            s_start(jj, 1, p)
            s_wait(jj, 1, p)
            g_start(jj + 2, 1, p)
            g_wait(0)
            s_start(jj + 1, 0, p)
            return carry

        lax.fori_loop(0, (CH - 2) // 2, inner, 0)
        s_wait(CH - 2, 0, p)
        if ch + 1 < nch:
            i_wait(1 - p)
            g_start(0, 0, 1 - p)
        g_wait(1)
        s_start(CH - 1, 1, p)

    s_wait(CH - 1, 1, (nch - 1) % 2)
    plsc.subcore_barrier()

    def w_in(k, b):
        pltpu.async_copy(acc_sp.at[pl.ds(s * RPT + k * G, G)], bufs[b],
                         semgs[b])

    def w_out(k, b):
        pltpu.async_copy(bufs[b], out.at[c, pl.ds(s * RPT + k * G, G)],
                         semss[b])

    def w_out_wait(k, b):
        pltpu.make_async_copy(
            bufs[b], out.at[c, pl.ds(s * RPT + k * G, G)], semss[b]).wait()

    w_in(0, 0)
    for k in range(nk):
        b = k % 2
        if k + 1 < nk:
            if k >= 1:
                w_out_wait(k - 1, 1 - b)
            w_in(k + 1, 1 - b)
        g_wait(b)
        w_out(k, b)
    w_out_wait(nk - 2, (nk - 2) % 2)
    w_out_wait(nk - 1, (nk - 1) % 2)


_agg_call = pl.kernel(
    _agg_body,
    out_type=jax.ShapeDtypeStruct((2, NP, 128), jnp.float32),
    mesh=_MESH,
    scratch_types=[
        pltpu.VMEM((CH, G), jnp.int32),
        pltpu.VMEM((CH, G), jnp.int32),
        pltpu.VMEM((CH, G), jnp.int32),
        pltpu.VMEM((CH, G), jnp.int32),
        pltpu.VMEM((G, 128), jnp.float32),
        pltpu.VMEM((G, 128), jnp.float32),
        pltpu.SemaphoreType.DMA,
        pltpu.SemaphoreType.DMA,
        pltpu.SemaphoreType.DMA,
        pltpu.SemaphoreType.DMA,
        pltpu.SemaphoreType.DMA,
        pltpu.SemaphoreType.DMA,
        pltpu.VMEM_SHARED((NP, 128), jnp.float32),
    ],
)


def _scale_body(feat_ref, deg_ref, fs_ref, dinv_ref):
    d = deg_ref[0, :] + deg_ref[1, :] + 1.0
    dinv = lax.rsqrt(d)[:, None]
    fs_ref[...] = feat_ref[...] * dinv
    dinv_ref[...] = dinv


_scale_call = pl.pallas_call(
    _scale_body,
    grid=(NP // BN,),
    in_specs=[
        pl.BlockSpec((BN, F_IN), lambda i: (i, 0)),
        pl.BlockSpec((2, BN), lambda i: (0, i)),
    ],
    out_specs=[
        pl.BlockSpec((BN, F_IN), lambda i: (i, 0)),
        pl.BlockSpec((BN, 1), lambda i: (i, 0)),
    ],
    out_shape=[
        jax.ShapeDtypeStruct((NP, F_IN), jnp.float32),
        jax.ShapeDtypeStruct((NP, 1), jnp.float32),
    ],
)


def _mm_body(agg_ref, fs_ref, dinv_ref, b1_ref, w1_ref, w2_ref, out_ref):
    x = agg_ref[0] + agg_ref[1] - fs_ref[...]
    dinv = dinv_ref[...]
    h1 = jnp.dot(x, w1_ref[...], preferred_element_type=jnp.float32)
    h = jnp.maximum(h1 * dinv + b1_ref[...], 0.0)
    h2 = jnp.dot(h, w2_ref[...], preferred_element_type=jnp.float32)
    out_ref[...] = h2 * dinv


_mm_call = pl.pallas_call(
    _mm_body,
    grid=(NP // BN,),
    in_specs=[
        pl.BlockSpec((2, BN, F_IN), lambda i: (0, i, 0)),
        pl.BlockSpec((BN, F_IN), lambda i: (i, 0)),
        pl.BlockSpec((BN, 1), lambda i: (i, 0)),
        pl.BlockSpec((1, HID), lambda i: (0, 0)),
        pl.BlockSpec((F_IN, HID), lambda i: (0, 0)),
        pl.BlockSpec((HID, 128), lambda i: (0, 0)),
    ],
    out_specs=pl.BlockSpec((BN, 128), lambda i: (i, 0)),
    out_shape=jax.ShapeDtypeStruct((NP, 128), jnp.float32),
)


def _head_body(agg_ref, hs2_ref, dinv_ref, b2_ref, out_ref):
    x = agg_ref[0, :, :64] + agg_ref[1, :, :64] - hs2_ref[:, :64]
    logits = x * dinv_ref[...] + b2_ref[...]
    m = jnp.max(logits, axis=1, keepdims=True)
    e = jnp.exp(logits - m)
    out_ref[...] = e / jnp.sum(e, axis=1, keepdims=True)


_head_call = pl.pallas_call(
    _head_body,
    grid=(NP // BN,),
    in_specs=[
        pl.BlockSpec((2, BN, 128), lambda i: (0, i, 0)),
        pl.BlockSpec((BN, 128), lambda i: (i, 0)),
        pl.BlockSpec((BN, 1), lambda i: (i, 0)),
        pl.BlockSpec((1, 64), lambda i: (0, 0)),
    ],
    out_specs=pl.BlockSpec((BN, 64), lambda i: (i, 0)),
    out_shape=jax.ShapeDtypeStruct((NP, 64), jnp.float32),
)


def kernel(feat, view, W1, b1, W2, b2):
    featp = jnp.zeros((NP, F_IN), jnp.float32).at[:N_NODES].set(feat)

    npad = EP - E_EDGES
    pad_i = jnp.arange(npad, dtype=jnp.int32)
    src = jnp.concatenate([view[0], pad_i % N_NODES])
    dst = jnp.concatenate([view[1], N_NODES + pad_i % (NP - N_NODES)])
    sidx2 = src.reshape(NGP, G)
    didx2 = dst.reshape(NGP, G)

    deg2 = _deg_call(didx2)
    fs, dinv = _scale_call(featp, deg2)
    agg1 = _agg_call(fs, sidx2, didx2)

    w2p = jnp.zeros((HID, 128), jnp.float32).at[:, :C_CLS].set(W2)
    hs2 = _mm_call(agg1, fs, dinv, b1.reshape(1, HID), W1, w2p)
    agg2 = _agg_call(hs2, sidx2, didx2)

    b2p = jnp.full((1, 64), -1e30, jnp.float32).at[0, :C_CLS].set(b2)
    prob = _head_call(agg2, hs2, dinv, b2p)
    return prob[:N_NODES, :C_CLS]


# speedup vs baseline: 31.6367x; 1.0011x over previous
import jax
import jax.numpy as jnp
from jax import lax
from jax.experimental import pallas as pl
from jax.experimental.pallas import tpu as pltpu
from jax.experimental.pallas import tpu_sc as plsc

N_NODES = 10000
E_EDGES = 320000
F_IN = 128
HID = 256
C_CLS = 40

NP = 10240
RPT = NP // 16
G = 128
NGP = 2560
CH = 20
EP = NGP * G
BN = 256

_MESH = plsc.VectorSubcoreMesh(core_axis_name="c", subcore_axis_name="s")


def _deg_body(didx2, deg_out, di0, di1, onesv, buf,
              semi0, semi1, sems0, sems1, deg_sp):
    c = lax.axis_index("c")
    s = lax.axis_index("s")
    gpt = NGP // 32
    nch = gpt // CH
    g0 = c * (NGP // 2) + s * gpt
    dib = (di0, di1)
    semis = (semi0, semi1)
    semss = (sems0, sems1)

    def i_start(ch, p):
        pltpu.async_copy(didx2.at[pl.ds(g0 + ch * CH, CH)], dib[p], semis[p])

    for i in range(G // 16):
        onesv[pl.ds(i * 16, 16)] = jnp.ones((16,), jnp.float32)
    i_start(0, 0)
    for i in range(RPT // 16):
        buf[pl.ds(i * 16, 16)] = jnp.zeros((16,), jnp.float32)
    pltpu.sync_copy(buf, deg_sp.at[pl.ds(s * RPT, RPT)])
    plsc.subcore_barrier()

    for ch in range(nch):
        p = ch % 2
        pltpu.make_async_copy(didx2.at[pl.ds(0, CH)], dib[p], semis[p]).wait()
        if ch > 0:
            for j in range(CH):
                pltpu.make_async_copy(
                    onesv, deg_sp.at[dib[1 - p].at[j]], semss[1 - p]).wait()
        if ch + 1 < nch:
            i_start(ch + 1, 1 - p)
        for j in range(CH):
            pltpu.async_copy(onesv, deg_sp.at[dib[p].at[j]], semss[p],
                             add=True)
    for j in range(CH):
        pltpu.make_async_copy(
            onesv, deg_sp.at[dib[(nch - 1) % 2].at[j]],
            semss[(nch - 1) % 2]).wait()
    plsc.subcore_barrier()
    pltpu.sync_copy(deg_sp.at[pl.ds(s * RPT, RPT)], buf)
    pltpu.sync_copy(buf, deg_out.at[c, pl.ds(s * RPT, RPT)])


_deg_call = pl.kernel(
    _deg_body,
    out_type=jax.ShapeDtypeStruct((2, NP), jnp.float32),
    mesh=_MESH,
    scratch_types=[
        pltpu.VMEM((CH, G), jnp.int32),
        pltpu.VMEM((CH, G), jnp.int32),
        pltpu.VMEM((G,), jnp.float32),
        pltpu.VMEM((RPT,), jnp.float32),
        pltpu.SemaphoreType.DMA,
        pltpu.SemaphoreType.DMA,
        pltpu.SemaphoreType.DMA,
        pltpu.SemaphoreType.DMA,
        pltpu.VMEM_SHARED((NP,), jnp.float32),
    ],
)


def _agg_body(table, sidx2, didx2, out,
              si0, si1, di0, di1, buf0, buf1,
              semi0, semi1, semg0, semg1, sems0, sems1, acc_sp):
    gpt = NGP // 32
    nch = gpt // CH
    c = lax.axis_index("c")
    s = lax.axis_index("s")
    gr0 = c * (NGP // 2) + s * gpt

    sib = (si0, si1)
    dib = (di0, di1)
    bufs = (buf0, buf1)
    semis = (semi0, semi1)
    semgs = (semg0, semg1)
    semss = (sems0, sems1)

    def i_start(ch, p):
        pltpu.async_copy(
            sidx2.at[pl.ds(gr0 + ch * CH, CH)], sib[p], semis[p])
        pltpu.async_copy(
            didx2.at[pl.ds(gr0 + ch * CH, CH)], dib[p], semis[p])

    def i_wait(p):
        pltpu.make_async_copy(
            sidx2.at[pl.ds(0, CH)], sib[p], semis[p]).wait()
        pltpu.make_async_copy(
            didx2.at[pl.ds(0, CH)], dib[p], semis[p]).wait()

    def g_start(j, b, p):
        pltpu.async_copy(table.at[sib[p].at[j]], bufs[b], semgs[b])

    def g_wait(b):
        pltpu.make_async_copy(table.at[pl.ds(0, G)], bufs[b], semgs[b]).wait()

    def s_start(j, b, p):
        pltpu.async_copy(bufs[b], acc_sp.at[dib[p].at[j]], semss[b], add=True)

    def s_wait(j, b, p):
        pltpu.make_async_copy(
            bufs[b], acc_sp.at[dib[p].at[j]], semss[b]).wait()

    i_start(0, 0)
    nk = RPT // G

    def seed_in(k, b):
        pltpu.async_copy(
            table.at[pl.ds(s * RPT + k * G, G)], bufs[b], semgs[b])

    def seed_out(k, b):
        pltpu.async_copy(bufs[b], acc_sp.at[pl.ds(s * RPT + k * G, G)],
                         semss[b])

    def seed_out_wait(k, b):
        pltpu.make_async_copy(
            bufs[b], acc_sp.at[pl.ds(s * RPT + k * G, G)], semss[b]).wait()

    seed_in(0, 0)
    for k in range(nk):
        b = k % 2
        if k + 1 < nk:
            if k >= 1:
                seed_out_wait(k - 1, 1 - b)
            seed_in(k + 1, 1 - b)
        g_wait(b)
        seed_out(k, b)
    seed_out_wait(nk - 2, (nk - 2) % 2)
    seed_out_wait(nk - 1, (nk - 1) % 2)
    plsc.subcore_barrier()

    for ch in range(nch):
        p = ch % 2
        if ch == 0:
            i_wait(0)
            if nch > 1:
                i_start(1, 1)
            g_start(0, 0, 0)
            g_start(1, 1, 0)
            g_wait(0)
            s_start(0, 0, 0)
        else:
            s_wait(CH - 1, 1, 1 - p)
            i_start(ch + 1, 1 - p) if ch + 1 < nch else None
            g_start(1, 1, p)
            g_wait(0)
            s_start(0, 0, p)

        def inner(j2, carry, p=p):
            jj = 2 * j2 + 1
            s_wait(jj - 1, 0, p)
            g_start(jj + 1, 0, p)
            g_wait(1)
            s_start(jj, 1, p)
            s_wait(jj, 1, p)
            g_start(jj + 2, 1, p)
            g_wait(0)
            s_start(jj + 1, 0, p)
            return carry

        lax.fori_loop(0, (CH - 2) // 2, inner, 0)
        s_wait(CH - 2, 0, p)
        if ch + 1 < nch:
            i_wait(1 - p)
            g_start(0, 0, 1 - p)
        g_wait(1)
        s_start(CH - 1, 1, p)

    s_wait(CH - 1, 1, (nch - 1) % 2)
    plsc.subcore_barrier()

    def w_in(k, b):
        pltpu.async_copy(acc_sp.at[pl.ds(s * RPT + k * G, G)], bufs[b],
                         semgs[b])

    def w_out(k, b):
        pltpu.async_copy(bufs[b], out.at[c, pl.ds(s * RPT + k * G, G)],
                         semss[b])

    def w_out_wait(k, b):
        pltpu.make_async_copy(
            bufs[b], out.at[c, pl.ds(s * RPT + k * G, G)], semss[b]).wait()

    w_in(0, 0)
    for k in range(nk):
        b = k % 2
        if k + 1 < nk:
            if k >= 1:
                w_out_wait(k - 1, 1 - b)
            w_in(k + 1, 1 - b)
        g_wait(b)
        w_out(k, b)
    w_out_wait(nk - 2, (nk - 2) % 2)
    w_out_wait(nk - 1, (nk - 1) % 2)


_agg_call = pl.kernel(
    _agg_body,
    out_type=jax.ShapeDtypeStruct((2, NP, 128), jnp.float32),
    mesh=_MESH,
    scratch_types=[
        pltpu.VMEM((CH, G), jnp.int32),
        pltpu.VMEM((CH, G), jnp.int32),
        pltpu.VMEM((CH, G), jnp.int32),
        pltpu.VMEM((CH, G), jnp.int32),
        pltpu.VMEM((G, 128), jnp.float32),
        pltpu.VMEM((G, 128), jnp.float32),
        pltpu.SemaphoreType.DMA,
        pltpu.SemaphoreType.DMA,
        pltpu.SemaphoreType.DMA,
        pltpu.SemaphoreType.DMA,
        pltpu.SemaphoreType.DMA,
        pltpu.SemaphoreType.DMA,
        pltpu.VMEM_SHARED((NP, 128), jnp.float32),
    ],
)


def _scale_body(feat_ref, deg_ref, fs_ref, dinv_ref):
    d = deg_ref[0, :] + deg_ref[1, :] + 1.0
    dinv = lax.rsqrt(d)[:, None]
    fs_ref[...] = feat_ref[...] * dinv
    dinv_ref[...] = dinv


_scale_call = pl.pallas_call(
    _scale_body,
    grid=(NP // BN,),
    in_specs=[
        pl.BlockSpec((BN, F_IN), lambda i: (i, 0)),
        pl.BlockSpec((2, BN), lambda i: (0, i)),
    ],
    out_specs=[
        pl.BlockSpec((BN, F_IN), lambda i: (i, 0)),
        pl.BlockSpec((BN, 1), lambda i: (i, 0)),
    ],
    out_shape=[
        jax.ShapeDtypeStruct((NP, F_IN), jnp.float32),
        jax.ShapeDtypeStruct((NP, 1), jnp.float32),
    ],
)


def _mm_body(agg_ref, fs_ref, dinv_ref, b1_ref, w1_ref, w2_ref, out_ref):
    x = agg_ref[0] + agg_ref[1] - fs_ref[...]
    dinv = dinv_ref[...]
    h1 = jnp.dot(x, w1_ref[...], preferred_element_type=jnp.float32)
    h = jnp.maximum(h1 * dinv + b1_ref[...], 0.0)
    h2 = jnp.dot(h, w2_ref[...], preferred_element_type=jnp.float32)
    out_ref[...] = h2 * dinv


_mm_call = pl.pallas_call(
    _mm_body,
    grid=(NP // BN,),
    in_specs=[
        pl.BlockSpec((2, BN, F_IN), lambda i: (0, i, 0)),
        pl.BlockSpec((BN, F_IN), lambda i: (i, 0)),
        pl.BlockSpec((BN, 1), lambda i: (i, 0)),
        pl.BlockSpec((1, HID), lambda i: (0, 0)),
        pl.BlockSpec((F_IN, HID), lambda i: (0, 0)),
        pl.BlockSpec((HID, 128), lambda i: (0, 0)),
    ],
    out_specs=pl.BlockSpec((BN, 128), lambda i: (i, 0)),
    out_shape=jax.ShapeDtypeStruct((NP, 128), jnp.float32),
)


def _head_body(agg_ref, hs2_ref, dinv_ref, b2_ref, out_ref):
    x = agg_ref[0, :, :64] + agg_ref[1, :, :64] - hs2_ref[:, :64]
    logits = x * dinv_ref[...] + b2_ref[...]
    m = jnp.max(logits, axis=1, keepdims=True)
    e = jnp.exp(logits - m)
    out_ref[...] = e / jnp.sum(e, axis=1, keepdims=True)


_head_call = pl.pallas_call(
    _head_body,
    grid=(NP // BN,),
    in_specs=[
        pl.BlockSpec((2, BN, 128), lambda i: (0, i, 0)),
        pl.BlockSpec((BN, 128), lambda i: (i, 0)),
        pl.BlockSpec((BN, 1), lambda i: (i, 0)),
        pl.BlockSpec((1, 64), lambda i: (0, 0)),
    ],
    out_specs=pl.BlockSpec((BN, 64), lambda i: (i, 0)),
    out_shape=jax.ShapeDtypeStruct((NP, 64), jnp.float32),
)


def kernel(feat, view, W1, b1, W2, b2):
    featp = jnp.zeros((NP, F_IN), jnp.float32).at[:N_NODES].set(feat)

    npad = EP - E_EDGES
    pad_i = jnp.arange(npad, dtype=jnp.int32)
    src = jnp.concatenate([view[0], pad_i % N_NODES])
    dst = jnp.concatenate([view[1], N_NODES + pad_i % (NP - N_NODES)])
    sidx2 = src.reshape(NGP, G)
    didx2 = dst.reshape(NGP, G)

    deg2 = _deg_call(didx2)
    fs, dinv = _scale_call(featp, deg2)
    agg1 = _agg_call(fs, sidx2, didx2)

    w2p = jnp.zeros((HID, 128), jnp.float32).at[:, :C_CLS].set(W2)
    hs2 = _mm_call(agg1, fs, dinv, b1.reshape(1, HID), W1, w2p)
    agg2 = _agg_call(hs2, sidx2, didx2)

    b2p = jnp.full((1, 64), -1e30, jnp.float32).at[0, :C_CLS].set(b2)
    prob = _head_call(agg2, hs2, dinv, b2p)
    return prob[:N_NODES, :C_CLS]
